# baseline (speedup 1.0000x reference)
import functools

import jax
import jax.numpy as jnp
from jax import lax
from jax.experimental import pallas as pl
from jax.experimental.pallas import tpu as pltpu
from jax.experimental.pallas import tpu_sc as plsc

NUM_TOKENS = 1000000
DIM = 64
BATCH = 4096
SEQ = 200

NW = 32
ST = SEQ // 8
BT = BATCH // 128
UNITS = ST * BT
PER_W = UNITS // NW


def _rsqrt(x):
    i = lax.bitcast_convert_type(x, jnp.int32)
    i = jnp.int32(0x5F3759DF) - lax.shift_right_logical(i, 1)
    y = lax.bitcast_convert_type(i, jnp.float32)
    for _ in range(3):
        y = y * (1.5 - 0.5 * x * y * y)
    return y


def _body(x_hbm, emb_hbm, out_hbm, idx_v, rows_v, ov, sem):
    wid = lax.axis_index("s") * 2 + lax.axis_index("c")
    lanes = lax.iota(jnp.int32, 16)

    @pl.loop(0, PER_W)
    def _unit(u):
        unit = wid * PER_W + u
        st = unit // BT
        bt = unit % BT
        pltpu.sync_copy(x_hbm.at[st, :, bt], idx_v)

        @pl.loop(0, 8)
        def _srow(ss):
            pltpu.async_copy(emb_hbm.at[idx_v.at[ss]], rows_v, sem).wait()

            for lg in range(8):
                rows_ids = lanes + (lg * 16)
                acc = None
                for d in range(64):
                    g = plsc.load_gather(
                        rows_v, [rows_ids, jnp.full((16,), d, jnp.int32)])
                    acc = g * g if acc is None else acc + g * g
                s = _rsqrt(jnp.maximum(acc, 1e-24))
                for dt in range(8):
                    for ds in range(8):
                        g = plsc.load_gather(
                            rows_v,
                            [rows_ids, jnp.full((16,), dt * 8 + ds, jnp.int32)])
                        ov[dt, ds, pl.ds(lg * 16, 16)] = g * s

            pltpu.sync_copy(ov, out_hbm.at[st * 8 + ss, :, bt])


@functools.partial(
    pl.kernel,
    out_type=jax.ShapeDtypeStruct((SEQ, 8, BT, 8, 128), jnp.float32),
    mesh=plsc.VectorSubcoreMesh(core_axis_name="c", subcore_axis_name="s"),
    compiler_params=pltpu.CompilerParams(
        needs_layout_passes=False, use_tc_tiling_on_sc=False),
    scratch_types=[
        pltpu.VMEM((8, 128), jnp.int32),
        pltpu.VMEM((128, DIM), jnp.float32),
        pltpu.VMEM((8, 8, 128), jnp.float32),
        pltpu.SemaphoreType.DMA,
    ],
)
def _embed_norm(x_hbm, emb_hbm, out_hbm, idx_v, rows_v, ov, sem):
    _body(x_hbm, emb_hbm, out_hbm, idx_v, rows_v, ov, sem)


def kernel(x, embed):
    x4 = jnp.transpose(x.astype(jnp.int32), (1, 0)).reshape(ST, 8, BT, 128)
    out5 = _embed_norm(x4, embed)
    return jnp.transpose(out5, (2, 4, 0, 1, 3)).reshape(BATCH, SEQ, DIM)


# speedup vs baseline: 1.0278x; 1.0278x over previous
import functools

import jax
import jax.numpy as jnp
from jax import lax
from jax.experimental import pallas as pl
from jax.experimental.pallas import tpu as pltpu
from jax.experimental.pallas import tpu_sc as plsc

NUM_TOKENS = 1000000
DIM = 64
BATCH = 4096
SEQ = 200

NW = 32
ST = SEQ // 8
BT = BATCH // 128
UNITS = ST * BT
PER_W = UNITS // NW


def _rsqrt(x):
    i = lax.bitcast_convert_type(x, jnp.int32)
    i = jnp.int32(0x5F3759DF) - lax.shift_right_logical(i, 1)
    y = lax.bitcast_convert_type(i, jnp.float32)
    for _ in range(3):
        y = y * (1.5 - 0.5 * x * y * y)
    return y


def _body(x_hbm, emb_hbm, out_hbm, idx_v, bufA, bufB, ovA, ovB,
          semA, semB, semOA, semOB):
    wid = lax.axis_index("s") * 2 + lax.axis_index("c")
    lanes = lax.iota(jnp.int32, 16)
    u0 = wid * PER_W
    st0 = u0 // BT
    bt0 = u0 % BT

    def fire_half(buf, h, sem):
        for j in range(4):
            pltpu.async_copy(
                emb_hbm.at[idx_v.at[h * 4 + j]],
                buf.at[pl.ds(j * 128, 128)], sem)

    def drain_half(buf, h, sem):
        for j in range(4):
            pltpu.make_async_copy(
                emb_hbm.at[idx_v.at[h * 4 + j]],
                buf.at[pl.ds(j * 128, 128)], sem).wait()

    def compute_pair(buf, r0, ov, sem_o, st, bt, s_lo):
        pltpu.make_async_copy(
            ov, out_hbm.at[pl.ds(s_lo, 2), :, bt], sem_o).wait()
        for rr in range(2):
            row_ids = [lanes + ((r0 + rr) * 128 + lg * 16) for lg in range(8)]

            @pl.loop(0, DIM, init_carry=tuple(
                jnp.zeros((16,), jnp.float32) for _ in range(8)))
            def _sumsq(d, acc):
                dv = jnp.broadcast_to(d, (16,))
                g = [plsc.load_gather(buf, [row_ids[lg], dv])
                     for lg in range(8)]
                return tuple(acc[lg] + g[lg] * g[lg] for lg in range(8))

            s8 = [_rsqrt(jnp.maximum(a, 1e-24)) for a in _sumsq]

            @pl.loop(0, DIM)
            def _scale(d):
                dv = jnp.broadcast_to(d, (16,))
                dt = d // 8
                ds_ = d % 8
                for lg in range(8):
                    g = plsc.load_gather(buf, [row_ids[lg], dv])
                    ov[rr, dt, ds_, pl.ds(lg * 16, 16)] = g * s8[lg]
        pltpu.async_copy(ov, out_hbm.at[pl.ds(s_lo, 2), :, bt], sem_o)

    pltpu.async_copy(ovA, out_hbm.at[pl.ds(st0 * 8, 2), :, bt0], semOA)
    pltpu.async_copy(ovB, out_hbm.at[pl.ds(st0 * 8 + 2, 2), :, bt0], semOB)
    pltpu.sync_copy(x_hbm.at[st0, :, bt0], idx_v)
    fire_half(bufA, 0, semA)

    @pl.loop(0, PER_W)
    def _unit(u):
        unit = u0 + u
        st = unit // BT
        bt = unit % BT
        fire_half(bufB, 1, semB)
        drain_half(bufA, 0, semA)
        compute_pair(bufA, 0, ovA, semOA, st, bt, st * 8)
        compute_pair(bufA, 2, ovB, semOB, st, bt, st * 8 + 2)
        drain_half(bufB, 1, semB)

        @pl.when(u + 1 < PER_W)
        def _prefetch():
            nunit = unit + 1
            pltpu.sync_copy(x_hbm.at[nunit // BT, :, nunit % BT], idx_v)
            fire_half(bufA, 0, semA)

        compute_pair(bufB, 0, ovA, semOA, st, bt, st * 8 + 4)
        compute_pair(bufB, 2, ovB, semOB, st, bt, st * 8 + 6)

    last = u0 + PER_W - 1
    lst, lbt = last // BT, last % BT
    pltpu.make_async_copy(
        ovA, out_hbm.at[pl.ds(lst * 8 + 4, 2), :, lbt], semOA).wait()
    pltpu.make_async_copy(
        ovB, out_hbm.at[pl.ds(lst * 8 + 6, 2), :, lbt], semOB).wait()


@functools.partial(
    pl.kernel,
    out_type=jax.ShapeDtypeStruct((SEQ, 8, BT, 8, 128), jnp.float32),
    mesh=plsc.VectorSubcoreMesh(core_axis_name="c", subcore_axis_name="s"),
    compiler_params=pltpu.CompilerParams(
        needs_layout_passes=False, use_tc_tiling_on_sc=False),
    scratch_types=[
        pltpu.VMEM((8, 128), jnp.int32),
        pltpu.VMEM((512, DIM), jnp.float32),
        pltpu.VMEM((512, DIM), jnp.float32),
        pltpu.VMEM((2, 8, 8, 128), jnp.float32),
        pltpu.VMEM((2, 8, 8, 128), jnp.float32),
        pltpu.SemaphoreType.DMA,
        pltpu.SemaphoreType.DMA,
        pltpu.SemaphoreType.DMA,
        pltpu.SemaphoreType.DMA,
    ],
)
def _embed_norm(x_hbm, emb_hbm, out_hbm, idx_v, bufA, bufB, ovA, ovB,
                semA, semB, semOA, semOB):
    _body(x_hbm, emb_hbm, out_hbm, idx_v, bufA, bufB, ovA, ovB,
          semA, semB, semOA, semOB)


def kernel(x, embed):
    x4 = jnp.transpose(x.astype(jnp.int32), (1, 0)).reshape(ST, 8, BT, 128)
    out5 = _embed_norm(x4, embed)
    return jnp.transpose(out5, (2, 4, 0, 1, 3)).reshape(BATCH, SEQ, DIM)


# speedup vs baseline: 1.0442x; 1.0160x over previous
import functools

import jax
import jax.numpy as jnp
from jax import lax
from jax.experimental import pallas as pl
from jax.experimental.pallas import tpu as pltpu
from jax.experimental.pallas import tpu_sc as plsc

NUM_TOKENS = 1000000
DIM = 64
BATCH = 4096
SEQ = 200

NW = 32
ST = SEQ // 8
BT = BATCH // 128
UNITS = ST * BT
PER_W = UNITS // NW


def _rsqrt(x):
    i = lax.bitcast_convert_type(x, jnp.int32)
    i = jnp.int32(0x5F3759DF) - lax.shift_right_logical(i, 1)
    y = lax.bitcast_convert_type(i, jnp.float32)
    for _ in range(3):
        y = y * (1.5 - 0.5 * x * y * y)
    return y


def _body(x_hbm, emb_hbm, out_hbm, idx_v, bufA, bufB, ovA, ovB,
          semA, semB, semOA, semOB):
    wid = lax.axis_index("s") * 2 + lax.axis_index("c")
    lanes = lax.iota(jnp.int32, 16)
    u0 = wid * PER_W
    st0 = u0 // BT
    bt0 = u0 % BT

    def fire_half(buf, h, sem):
        for j in range(4):
            pltpu.async_copy(
                emb_hbm.at[idx_v.at[h * 4 + j]],
                buf.at[pl.ds(j * 128, 128)], sem)

    def drain_half(buf, h, sem):
        for j in range(4):
            pltpu.make_async_copy(
                emb_hbm.at[idx_v.at[h * 4 + j]],
                buf.at[pl.ds(j * 128, 128)], sem).wait()

    def compute_pair(buf, r0, ov, sem_o, st, bt, s_lo):
        pltpu.make_async_copy(
            ov, out_hbm.at[pl.ds(s_lo, 2), :, bt], sem_o).wait()
        for rr in range(2):
            row_ids = [lanes + ((r0 + rr) * 128 + lg * 16) for lg in range(8)]

            @pl.loop(0, DIM, init_carry=tuple(
                jnp.zeros((16,), jnp.float32) for _ in range(8)), unroll=8)
            def _sumsq(d, acc):
                dv = jnp.broadcast_to(d, (16,))
                g = [plsc.load_gather(buf, [row_ids[lg], dv])
                     for lg in range(8)]
                return tuple(acc[lg] + g[lg] * g[lg] for lg in range(8))

            s8 = [_rsqrt(jnp.maximum(a, 1e-24)) for a in _sumsq]

            @pl.loop(0, DIM, unroll=8)
            def _scale(d):
                dv = jnp.broadcast_to(d, (16,))
                dt = d // 8
                ds_ = d % 8
                for lg in range(8):
                    g = plsc.load_gather(buf, [row_ids[lg], dv])
                    ov[rr, dt, ds_, pl.ds(lg * 16, 16)] = g * s8[lg]
        pltpu.async_copy(ov, out_hbm.at[pl.ds(s_lo, 2), :, bt], sem_o)

    pltpu.async_copy(ovA, out_hbm.at[pl.ds(st0 * 8, 2), :, bt0], semOA)
    pltpu.async_copy(ovB, out_hbm.at[pl.ds(st0 * 8 + 2, 2), :, bt0], semOB)
    pltpu.sync_copy(x_hbm.at[st0, :, bt0], idx_v)
    fire_half(bufA, 0, semA)

    @pl.loop(0, PER_W)
    def _unit(u):
        unit = u0 + u
        st = unit // BT
        bt = unit % BT
        fire_half(bufB, 1, semB)
        drain_half(bufA, 0, semA)
        compute_pair(bufA, 0, ovA, semOA, st, bt, st * 8)
        compute_pair(bufA, 2, ovB, semOB, st, bt, st * 8 + 2)
        drain_half(bufB, 1, semB)

        @pl.when(u + 1 < PER_W)
        def _prefetch():
            nunit = unit + 1
            pltpu.sync_copy(x_hbm.at[nunit // BT, :, nunit % BT], idx_v)
            fire_half(bufA, 0, semA)

        compute_pair(bufB, 0, ovA, semOA, st, bt, st * 8 + 4)
        compute_pair(bufB, 2, ovB, semOB, st, bt, st * 8 + 6)

    last = u0 + PER_W - 1
    lst, lbt = last // BT, last % BT
    pltpu.make_async_copy(
        ovA, out_hbm.at[pl.ds(lst * 8 + 4, 2), :, lbt], semOA).wait()
    pltpu.make_async_copy(
        ovB, out_hbm.at[pl.ds(lst * 8 + 6, 2), :, lbt], semOB).wait()


@functools.partial(
    pl.kernel,
    out_type=jax.ShapeDtypeStruct((SEQ, 8, BT, 8, 128), jnp.float32),
    mesh=plsc.VectorSubcoreMesh(core_axis_name="c", subcore_axis_name="s"),
    compiler_params=pltpu.CompilerParams(
        needs_layout_passes=False, use_tc_tiling_on_sc=False),
    scratch_types=[
        pltpu.VMEM((8, 128), jnp.int32),
        pltpu.VMEM((512, DIM), jnp.float32),
        pltpu.VMEM((512, DIM), jnp.float32),
        pltpu.VMEM((2, 8, 8, 128), jnp.float32),
        pltpu.VMEM((2, 8, 8, 128), jnp.float32),
        pltpu.SemaphoreType.DMA,
        pltpu.SemaphoreType.DMA,
        pltpu.SemaphoreType.DMA,
        pltpu.SemaphoreType.DMA,
    ],
)
def _embed_norm(x_hbm, emb_hbm, out_hbm, idx_v, bufA, bufB, ovA, ovB,
                semA, semB, semOA, semOB):
    _body(x_hbm, emb_hbm, out_hbm, idx_v, bufA, bufB, ovA, ovB,
          semA, semB, semOA, semOB)


def kernel(x, embed):
    x4 = jnp.transpose(x.astype(jnp.int32), (1, 0)).reshape(ST, 8, BT, 128)
    out5 = _embed_norm(x4, embed)
    return jnp.transpose(out5, (2, 4, 0, 1, 3)).reshape(BATCH, SEQ, DIM)


# speedup vs baseline: 1.3725x; 1.3143x over previous
import functools

import jax
import jax.numpy as jnp
from jax import lax
from jax.experimental import pallas as pl
from jax.experimental.pallas import tpu as pltpu
from jax.experimental.pallas import tpu_sc as plsc

NUM_TOKENS = 1000000
DIM = 64
BATCH = 4096
SEQ = 200

NW = 32
ST = SEQ // 8
BT = BATCH // 128
UNITS = ST * BT
PER_W = UNITS // NW


def _rsqrt(x):
    i = lax.bitcast_convert_type(x, jnp.int32)
    i = jnp.int32(0x5F3759DF) - lax.shift_right_logical(i, 1)
    y = lax.bitcast_convert_type(i, jnp.float32)
    for _ in range(3):
        y = y * (1.5 - 0.5 * x * y * y)
    return y


def _body(x_hbm, emb_hbm, out_hbm, idx_v, bufA, bufB, ovA, ovB,
          semA, semB, semOA, semOB):
    wid = lax.axis_index("s") * 2 + lax.axis_index("c")
    lanes = lax.iota(jnp.int32, 16)
    u0 = wid * PER_W
    st0 = u0 // BT
    bt0 = u0 % BT

    def fire_half(buf, h, sem):
        for j in range(4):
            pltpu.async_copy(
                emb_hbm.at[idx_v.at[h * 4 + j]],
                buf.at[pl.ds(j * 128, 128)], sem)

    def drain_half(buf, h, sem):
        for j in range(4):
            pltpu.make_async_copy(
                emb_hbm.at[idx_v.at[h * 4 + j]],
                buf.at[pl.ds(j * 128, 128)], sem).wait()

    def compute_pair(buf, r0, ov, sem_o, st, bt, s_lo):
        pltpu.make_async_copy(
            ov, out_hbm.at[pl.ds(s_lo, 2), :, bt], sem_o).wait()
        for rr in range(2):
            row_ids = [lanes + ((r0 + rr) * 128 + lg * 16) for lg in range(8)]

            @pl.loop(0, DIM, init_carry=tuple(
                jnp.zeros((16,), jnp.float32) for _ in range(8)), unroll=8)
            def _sumsq(d, acc):
                dv = (jnp.broadcast_to(d, (16,)) + lanes) & 63
                g = [plsc.load_gather(buf, [row_ids[lg], dv])
                     for lg in range(8)]
                return tuple(acc[lg] + g[lg] * g[lg] for lg in range(8))

            s8 = [_rsqrt(jnp.maximum(a, 1e-24)) for a in _sumsq]

            @pl.loop(0, DIM, unroll=8)
            def _scale(d):
                dv = jnp.broadcast_to(d, (16,))
                dt = d // 8
                ds_ = d % 8
                for lg in range(8):
                    g = plsc.load_gather(buf, [row_ids[lg], dv])
                    ov[rr, dt, ds_, pl.ds(lg * 16, 16)] = g * s8[lg]
        pltpu.async_copy(ov, out_hbm.at[pl.ds(s_lo, 2), :, bt], sem_o)

    pltpu.async_copy(ovA, out_hbm.at[pl.ds(st0 * 8, 2), :, bt0], semOA)
    pltpu.async_copy(ovB, out_hbm.at[pl.ds(st0 * 8 + 2, 2), :, bt0], semOB)
    pltpu.sync_copy(x_hbm.at[st0, :, bt0], idx_v)
    fire_half(bufA, 0, semA)

    @pl.loop(0, PER_W)
    def _unit(u):
        unit = u0 + u
        st = unit // BT
        bt = unit % BT
        fire_half(bufB, 1, semB)
        drain_half(bufA, 0, semA)
        compute_pair(bufA, 0, ovA, semOA, st, bt, st * 8)
        compute_pair(bufA, 2, ovB, semOB, st, bt, st * 8 + 2)
        drain_half(bufB, 1, semB)

        @pl.when(u + 1 < PER_W)
        def _prefetch():
            nunit = unit + 1
            pltpu.sync_copy(x_hbm.at[nunit // BT, :, nunit % BT], idx_v)
            fire_half(bufA, 0, semA)

        compute_pair(bufB, 0, ovA, semOA, st, bt, st * 8 + 4)
        compute_pair(bufB, 2, ovB, semOB, st, bt, st * 8 + 6)

    last = u0 + PER_W - 1
    lst, lbt = last // BT, last % BT
    pltpu.make_async_copy(
        ovA, out_hbm.at[pl.ds(lst * 8 + 4, 2), :, lbt], semOA).wait()
    pltpu.make_async_copy(
        ovB, out_hbm.at[pl.ds(lst * 8 + 6, 2), :, lbt], semOB).wait()


@functools.partial(
    pl.kernel,
    out_type=jax.ShapeDtypeStruct((SEQ, 8, BT, 8, 128), jnp.float32),
    mesh=plsc.VectorSubcoreMesh(core_axis_name="c", subcore_axis_name="s"),
    compiler_params=pltpu.CompilerParams(
        needs_layout_passes=False, use_tc_tiling_on_sc=False),
    scratch_types=[
        pltpu.VMEM((8, 128), jnp.int32),
        pltpu.VMEM((512, DIM), jnp.float32),
        pltpu.VMEM((512, DIM), jnp.float32),
        pltpu.VMEM((2, 8, 8, 128), jnp.float32),
        pltpu.VMEM((2, 8, 8, 128), jnp.float32),
        pltpu.SemaphoreType.DMA,
        pltpu.SemaphoreType.DMA,
        pltpu.SemaphoreType.DMA,
        pltpu.SemaphoreType.DMA,
    ],
)
def _embed_norm(x_hbm, emb_hbm, out_hbm, idx_v, bufA, bufB, ovA, ovB,
                semA, semB, semOA, semOB):
    _body(x_hbm, emb_hbm, out_hbm, idx_v, bufA, bufB, ovA, ovB,
          semA, semB, semOA, semOB)


def kernel(x, embed):
    x4 = jnp.transpose(x.astype(jnp.int32), (1, 0)).reshape(ST, 8, BT, 128)
    out5 = _embed_norm(x4, embed)
    return jnp.transpose(out5, (2, 4, 0, 1, 3)).reshape(BATCH, SEQ, DIM)


# speedup vs baseline: 2.3544x; 1.7154x over previous
import functools

import jax
import jax.numpy as jnp
from jax import lax
from jax.experimental import pallas as pl
from jax.experimental.pallas import tpu as pltpu
from jax.experimental.pallas import tpu_sc as plsc

NUM_TOKENS = 1000000
DIM = 64
BATCH = 4096
SEQ = 200

NW = 32
ST = SEQ // 8
BT = BATCH // 128
UNITS = ST * BT
PER_W = UNITS // NW


def _rsqrt(x):
    i = lax.bitcast_convert_type(x, jnp.int32)
    i = jnp.int32(0x5F3759DF) - lax.shift_right_logical(i, 1)
    y = lax.bitcast_convert_type(i, jnp.float32)
    for _ in range(3):
        y = y * (1.5 - 0.5 * x * y * y)
    return y


def _body(x_hbm, emb_hbm, out_hbm, idx_v, bufA, bufB, ovA, ovB,
          semA, semB, semOA, semOB):
    wid = lax.axis_index("s") * 2 + lax.axis_index("c")
    lanes = lax.iota(jnp.int32, 16)
    u0 = wid * PER_W
    st0 = u0 // BT
    bt0 = u0 % BT

    def fire_half(buf, h, sem):
        for j in range(4):
            pltpu.async_copy(
                emb_hbm.at[idx_v.at[h * 4 + j]],
                buf.at[pl.ds(j * 128, 128)], sem)

    def drain_half(buf, h, sem):
        for j in range(4):
            pltpu.make_async_copy(
                emb_hbm.at[idx_v.at[h * 4 + j]],
                buf.at[pl.ds(j * 128, 128)], sem).wait()

    def compute_pair(buf, r0, ov, sem_o, st, bt, s_lo):
        pltpu.make_async_copy(
            ov, out_hbm.at[pl.ds(s_lo, 2), :, bt], sem_o).wait()
        for rr in range(2):
            row_ids = [lanes + ((r0 + rr) * 128 + lg * 16) for lg in range(8)]

            @pl.loop(0, DIM, init_carry=tuple(
                jnp.zeros((16,), jnp.float32) for _ in range(8)), unroll=8)
            def _sumsq(d, acc):
                dv = (jnp.broadcast_to(d, (16,)) + lanes) & 63
                g = [plsc.load_gather(buf, [row_ids[lg], dv])
                     for lg in range(8)]
                return tuple(acc[lg] + g[lg] * g[lg] for lg in range(8))

            s8 = [_rsqrt(jnp.maximum(a, 1e-24)) for a in _sumsq]

            @pl.loop(0, DIM, unroll=8)
            def _scale(d):
                c = (jnp.broadcast_to(d, (16,)) + lanes) & 63
                i0 = jnp.full((16,), rr, jnp.int32)
                i1 = c >> 3
                i2 = c & 7
                for lg in range(8):
                    g = plsc.load_gather(buf, [row_ids[lg], c])
                    plsc.store_scatter(
                        ov, [i0, i1, i2, lanes + lg * 16], g * s8[lg])
        pltpu.async_copy(ov, out_hbm.at[pl.ds(s_lo, 2), :, bt], sem_o)

    pltpu.async_copy(ovA, out_hbm.at[pl.ds(st0 * 8, 2), :, bt0], semOA)
    pltpu.async_copy(ovB, out_hbm.at[pl.ds(st0 * 8 + 2, 2), :, bt0], semOB)
    pltpu.sync_copy(x_hbm.at[st0, :, bt0], idx_v)
    fire_half(bufA, 0, semA)

    @pl.loop(0, PER_W)
    def _unit(u):
        unit = u0 + u
        st = unit // BT
        bt = unit % BT
        fire_half(bufB, 1, semB)
        drain_half(bufA, 0, semA)
        compute_pair(bufA, 0, ovA, semOA, st, bt, st * 8)
        compute_pair(bufA, 2, ovB, semOB, st, bt, st * 8 + 2)
        drain_half(bufB, 1, semB)

        @pl.when(u + 1 < PER_W)
        def _prefetch():
            nunit = unit + 1
            pltpu.sync_copy(x_hbm.at[nunit // BT, :, nunit % BT], idx_v)
            fire_half(bufA, 0, semA)

        compute_pair(bufB, 0, ovA, semOA, st, bt, st * 8 + 4)
        compute_pair(bufB, 2, ovB, semOB, st, bt, st * 8 + 6)

    last = u0 + PER_W - 1
    lst, lbt = last // BT, last % BT
    pltpu.make_async_copy(
        ovA, out_hbm.at[pl.ds(lst * 8 + 4, 2), :, lbt], semOA).wait()
    pltpu.make_async_copy(
        ovB, out_hbm.at[pl.ds(lst * 8 + 6, 2), :, lbt], semOB).wait()


@functools.partial(
    pl.kernel,
    out_type=jax.ShapeDtypeStruct((SEQ, 8, BT, 8, 128), jnp.float32),
    mesh=plsc.VectorSubcoreMesh(core_axis_name="c", subcore_axis_name="s"),
    compiler_params=pltpu.CompilerParams(
        needs_layout_passes=False, use_tc_tiling_on_sc=False),
    scratch_types=[
        pltpu.VMEM((8, 128), jnp.int32),
        pltpu.VMEM((512, DIM), jnp.float32),
        pltpu.VMEM((512, DIM), jnp.float32),
        pltpu.VMEM((2, 8, 8, 128), jnp.float32),
        pltpu.VMEM((2, 8, 8, 128), jnp.float32),
        pltpu.SemaphoreType.DMA,
        pltpu.SemaphoreType.DMA,
        pltpu.SemaphoreType.DMA,
        pltpu.SemaphoreType.DMA,
    ],
)
def _embed_norm(x_hbm, emb_hbm, out_hbm, idx_v, bufA, bufB, ovA, ovB,
                semA, semB, semOA, semOB):
    _body(x_hbm, emb_hbm, out_hbm, idx_v, bufA, bufB, ovA, ovB,
          semA, semB, semOA, semOB)


def kernel(x, embed):
    x4 = jnp.transpose(x.astype(jnp.int32), (1, 0)).reshape(ST, 8, BT, 128)
    out5 = _embed_norm(x4, embed)
    return jnp.transpose(out5, (2, 4, 0, 1, 3)).reshape(BATCH, SEQ, DIM)


# speedup vs baseline: 2.3609x; 1.0028x over previous
import functools

import jax
import jax.numpy as jnp
from jax import lax
from jax.experimental import pallas as pl
from jax.experimental.pallas import tpu as pltpu
from jax.experimental.pallas import tpu_sc as plsc

NUM_TOKENS = 1000000
DIM = 64
BATCH = 4096
SEQ = 200

NW = 32
ST = SEQ // 8
BT = BATCH // 128
UNITS = ST * BT
PER_W = UNITS // NW


def _rsqrt(x):
    i = lax.bitcast_convert_type(x, jnp.int32)
    i = jnp.int32(0x5F3759DF) - lax.shift_right_logical(i, 1)
    y = lax.bitcast_convert_type(i, jnp.float32)
    for _ in range(3):
        y = y * (1.5 - 0.5 * x * y * y)
    return y


def _body(x_hbm, emb_hbm, out_hbm, idx_v, bufA, bufB, ovA, ovB,
          semA, semB, semOA, semOB):
    wid = lax.axis_index("s") * 2 + lax.axis_index("c")
    lanes = lax.iota(jnp.int32, 16)
    rot = lanes * 4 + (lanes >> 2)
    u0 = wid * PER_W
    st0 = u0 // BT
    bt0 = u0 % BT

    def fire_half(buf, h, sem):
        for j in range(4):
            pltpu.async_copy(
                emb_hbm.at[idx_v.at[h * 4 + j]],
                buf.at[pl.ds(j * 128, 128)], sem)

    def drain_half(buf, h, sem):
        for j in range(4):
            pltpu.make_async_copy(
                emb_hbm.at[idx_v.at[h * 4 + j]],
                buf.at[pl.ds(j * 128, 128)], sem).wait()

    def compute_pair(buf, r0, ov, sem_o, st, bt, s_lo):
        pltpu.make_async_copy(
            ov, out_hbm.at[pl.ds(s_lo, 2), :, bt], sem_o).wait()
        for rr in range(2):
            row_ids = [lanes + ((r0 + rr) * 128 + lg * 16) for lg in range(8)]

            @pl.loop(0, DIM, init_carry=tuple(
                jnp.zeros((16,), jnp.float32) for _ in range(8)), unroll=8)
            def _sumsq(d, acc):
                dv = (jnp.broadcast_to(d, (16,)) + rot) & 63
                g = [plsc.load_gather(buf, [row_ids[lg], dv])
                     for lg in range(8)]
                return tuple(acc[lg] + g[lg] * g[lg] for lg in range(8))

            s8 = [_rsqrt(jnp.maximum(a, 1e-24)) for a in _sumsq]

            @pl.loop(0, DIM, unroll=8)
            def _scale(d):
                c = (jnp.broadcast_to(d, (16,)) + rot) & 63
                i0 = jnp.full((16,), rr, jnp.int32)
                i1 = c >> 3
                i2 = c & 7
                for lg in range(8):
                    g = plsc.load_gather(buf, [row_ids[lg], c])
                    plsc.store_scatter(
                        ov, [i0, i1, i2, lanes + lg * 16], g * s8[lg])
        pltpu.async_copy(ov, out_hbm.at[pl.ds(s_lo, 2), :, bt], sem_o)

    pltpu.async_copy(ovA, out_hbm.at[pl.ds(st0 * 8, 2), :, bt0], semOA)
    pltpu.async_copy(ovB, out_hbm.at[pl.ds(st0 * 8 + 2, 2), :, bt0], semOB)
    pltpu.sync_copy(x_hbm.at[st0, :, bt0], idx_v)
    fire_half(bufA, 0, semA)

    @pl.loop(0, PER_W)
    def _unit(u):
        unit = u0 + u
        st = unit // BT
        bt = unit % BT
        fire_half(bufB, 1, semB)
        drain_half(bufA, 0, semA)
        compute_pair(bufA, 0, ovA, semOA, st, bt, st * 8)
        compute_pair(bufA, 2, ovB, semOB, st, bt, st * 8 + 2)
        drain_half(bufB, 1, semB)

        @pl.when(u + 1 < PER_W)
        def _prefetch():
            nunit = unit + 1
            pltpu.sync_copy(x_hbm.at[nunit // BT, :, nunit % BT], idx_v)
            fire_half(bufA, 0, semA)

        compute_pair(bufB, 0, ovA, semOA, st, bt, st * 8 + 4)
        compute_pair(bufB, 2, ovB, semOB, st, bt, st * 8 + 6)

    last = u0 + PER_W - 1
    lst, lbt = last // BT, last % BT
    pltpu.make_async_copy(
        ovA, out_hbm.at[pl.ds(lst * 8 + 4, 2), :, lbt], semOA).wait()
    pltpu.make_async_copy(
        ovB, out_hbm.at[pl.ds(lst * 8 + 6, 2), :, lbt], semOB).wait()


@functools.partial(
    pl.kernel,
    out_type=jax.ShapeDtypeStruct((SEQ, 8, BT, 8, 128), jnp.float32),
    mesh=plsc.VectorSubcoreMesh(core_axis_name="c", subcore_axis_name="s"),
    compiler_params=pltpu.CompilerParams(
        needs_layout_passes=False, use_tc_tiling_on_sc=False),
    scratch_types=[
        pltpu.VMEM((8, 128), jnp.int32),
        pltpu.VMEM((512, DIM), jnp.float32),
        pltpu.VMEM((512, DIM), jnp.float32),
        pltpu.VMEM((2, 8, 8, 128), jnp.float32),
        pltpu.VMEM((2, 8, 8, 128), jnp.float32),
        pltpu.SemaphoreType.DMA,
        pltpu.SemaphoreType.DMA,
        pltpu.SemaphoreType.DMA,
        pltpu.SemaphoreType.DMA,
    ],
)
def _embed_norm(x_hbm, emb_hbm, out_hbm, idx_v, bufA, bufB, ovA, ovB,
                semA, semB, semOA, semOB):
    _body(x_hbm, emb_hbm, out_hbm, idx_v, bufA, bufB, ovA, ovB,
          semA, semB, semOA, semOB)


def kernel(x, embed):
    x4 = jnp.transpose(x.astype(jnp.int32), (1, 0)).reshape(ST, 8, BT, 128)
    out5 = _embed_norm(x4, embed)
    return jnp.transpose(out5, (2, 4, 0, 1, 3)).reshape(BATCH, SEQ, DIM)


# speedup vs baseline: 2.3647x; 1.0016x over previous
import functools

import jax
import jax.numpy as jnp
from jax import lax
from jax.experimental import pallas as pl
from jax.experimental.pallas import tpu as pltpu
from jax.experimental.pallas import tpu_sc as plsc

NUM_TOKENS = 1000000
DIM = 64
BATCH = 4096
SEQ = 200

NW = 32
ST = SEQ // 8
BT = BATCH // 128
UNITS = ST * BT
PER_W = UNITS // NW


def _rsqrt(x):
    i = lax.bitcast_convert_type(x, jnp.int32)
    i = jnp.int32(0x5F3759DF) - lax.shift_right_logical(i, 1)
    y = lax.bitcast_convert_type(i, jnp.float32)
    for _ in range(3):
        y = y * (1.5 - 0.5 * x * y * y)
    return y


def _body(x_hbm, emb_hbm, out_hbm, idx_v, bufA, bufB, ovA, ovB,
          semA, semB, semOA, semOB):
    wid = lax.axis_index("s") * 2 + lax.axis_index("c")
    lanes = lax.iota(jnp.int32, 16)
    rot = lanes * 4 + (lanes >> 2)
    u0 = wid * PER_W
    st0 = u0 // BT
    bt0 = u0 % BT

    def fire_half(buf, h, sem):
        for j in range(4):
            pltpu.async_copy(
                emb_hbm.at[idx_v.at[h * 4 + j]],
                buf.at[pl.ds(j * 128, 128)], sem)

    def drain_half(buf, h, sem):
        for j in range(4):
            pltpu.make_async_copy(
                emb_hbm.at[idx_v.at[h * 4 + j]],
                buf.at[pl.ds(j * 128, 128)], sem).wait()

    def compute_pair(buf, r0, ov, sem_o, st, bt, s_lo):
        pltpu.make_async_copy(
            ov, out_hbm.at[pl.ds(s_lo, 2), :, bt], sem_o).wait()
        for rr in range(2):
            row_ids = [lanes + ((r0 + rr) * 128 + lg * 16) for lg in range(8)]

            @pl.loop(0, DIM, init_carry=tuple(
                jnp.zeros((16,), jnp.float32) for _ in range(8)), unroll=8)
            def _sumsq(d, acc):
                dv = (jnp.broadcast_to(d, (16,)) + rot) & 63
                g = [plsc.load_gather(buf, [row_ids[lg], dv])
                     for lg in range(8)]
                return tuple(acc[lg] + g[lg] * g[lg] for lg in range(8))

            s8 = [_rsqrt(jnp.maximum(a, 1e-24)) for a in _sumsq]

            @pl.loop(0, DIM, unroll=8)
            def _scale(d):
                c = (jnp.broadcast_to(d, (16,)) + rot) & 63
                i0 = jnp.full((16,), rr, jnp.int32)
                i1 = c >> 3
                i2 = c & 7
                for lg in range(8):
                    g = plsc.load_gather(buf, [row_ids[lg], c])
                    plsc.store_scatter(
                        ov, [i0, i1, i2, lanes + lg * 16], g * s8[lg])
        pltpu.async_copy(ov, out_hbm.at[pl.ds(s_lo, 2), :, bt], sem_o)

    pltpu.async_copy(ovA, out_hbm.at[pl.ds(st0 * 8, 2), :, bt0], semOA)
    pltpu.async_copy(ovB, out_hbm.at[pl.ds(st0 * 8 + 2, 2), :, bt0], semOB)
    pltpu.sync_copy(x_hbm.at[st0, bt0], idx_v)
    fire_half(bufA, 0, semA)

    @pl.loop(0, PER_W)
    def _unit(u):
        unit = u0 + u
        st = unit // BT
        bt = unit % BT
        fire_half(bufB, 1, semB)
        drain_half(bufA, 0, semA)
        compute_pair(bufA, 0, ovA, semOA, st, bt, st * 8)
        compute_pair(bufA, 2, ovB, semOB, st, bt, st * 8 + 2)
        drain_half(bufB, 1, semB)

        @pl.when(u + 1 < PER_W)
        def _prefetch():
            nunit = unit + 1
            pltpu.sync_copy(x_hbm.at[nunit // BT, nunit % BT], idx_v)
            fire_half(bufA, 0, semA)

        compute_pair(bufB, 0, ovA, semOA, st, bt, st * 8 + 4)
        compute_pair(bufB, 2, ovB, semOB, st, bt, st * 8 + 6)

    last = u0 + PER_W - 1
    lst, lbt = last // BT, last % BT
    pltpu.make_async_copy(
        ovA, out_hbm.at[pl.ds(lst * 8 + 4, 2), :, lbt], semOA).wait()
    pltpu.make_async_copy(
        ovB, out_hbm.at[pl.ds(lst * 8 + 6, 2), :, lbt], semOB).wait()


@functools.partial(
    pl.kernel,
    out_type=jax.ShapeDtypeStruct((SEQ, 8, BT, 8, 128), jnp.float32),
    mesh=plsc.VectorSubcoreMesh(core_axis_name="c", subcore_axis_name="s"),
    compiler_params=pltpu.CompilerParams(
        needs_layout_passes=False, use_tc_tiling_on_sc=False),
    scratch_types=[
        pltpu.VMEM((8, 128), jnp.int32),
        pltpu.VMEM((512, DIM), jnp.float32),
        pltpu.VMEM((512, DIM), jnp.float32),
        pltpu.VMEM((2, 8, 8, 128), jnp.float32),
        pltpu.VMEM((2, 8, 8, 128), jnp.float32),
        pltpu.SemaphoreType.DMA,
        pltpu.SemaphoreType.DMA,
        pltpu.SemaphoreType.DMA,
        pltpu.SemaphoreType.DMA,
    ],
)
def _embed_norm(x_hbm, emb_hbm, out_hbm, idx_v, bufA, bufB, ovA, ovB,
                semA, semB, semOA, semOB):
    _body(x_hbm, emb_hbm, out_hbm, idx_v, bufA, bufB, ovA, ovB,
          semA, semB, semOA, semOB)


def kernel(x, embed):
    x4 = jnp.transpose(
        jnp.transpose(x.astype(jnp.int32), (1, 0)).reshape(ST, 8, BT, 128),
        (0, 2, 1, 3))
    out5 = _embed_norm(x4, embed)
    return jnp.transpose(out5, (2, 4, 0, 1, 3)).reshape(BATCH, SEQ, DIM)
